# Initial kernel scaffold; baseline (speedup 1.0000x reference)
import jax
import jax.numpy as jnp
from jax.experimental import pallas as pl


def kernel(x, alpha, delta, r):
    raise NotImplementedError("write your pallas kernel here")


# speedup vs baseline: 2.8263x; 2.8263x over previous
import functools

import numpy as np
import jax
import jax.numpy as jnp
from jax.experimental import pallas as pl
from jax.experimental.pallas import tpu as pltpu

_T_VAL = 256.0
_EPS = 1e-05
_W = 256
_BB = 2


def _smoothing_coef() -> float:
    return float((np.sqrt(1.0 + 4.0 * _T_VAL ** 2) - 1.0) / (2.0 * _T_VAL ** 2))


@functools.lru_cache(maxsize=None)
def _tables(n_bands: int):
    s = _smoothing_coef()
    j = np.arange(_W)
    m = j[:, None] - j[None, :]
    vt = np.where(m >= 0, s * (1.0 - s) ** np.maximum(m, 0), 0.0)
    a_mat = np.concatenate(
        [vt, np.eye(_W), vt[-1:, :], np.zeros((7, _W))], axis=0)
    q_full = np.tile(((1.0 - s) ** (j + 1))[:, None], (1, _BB * n_bands))
    return (jnp.asarray(a_mat, dtype=jnp.float32),
            jnp.asarray(q_full, dtype=jnp.float32),
            float((1.0 - s) ** _W))


def _pcen_body(x_ref, a_ref, q_ref, alpha_ref, delta_ref, r_ref,
               out_ref, carry_ref, *, decay):
    t = pl.program_id(1)
    nb = x_ref.shape[2]
    w = q_ref.shape[0]
    x2 = x_ref[...].reshape(_BB * nb, w)
    res = jax.lax.dot_general(
        a_ref[...], x2, (((1,), (1,)), ((), ())),
        preferred_element_type=jnp.float32)
    x_t = res[w:2 * w, :]

    @pl.when(t == 0)
    def _():
        carry_ref[...] = x_t[0:1, :]

    carry = carry_ref[...]
    smoother = res[:w, :] + q_ref[...] * carry
    carry_ref[...] = res[2 * w:2 * w + 1, :] + decay * carry

    a = jnp.exp(alpha_ref[...])
    d = jnp.exp(delta_ref[...])
    rr = jnp.exp(r_ref[...])
    drr = jnp.exp(rr * delta_ref[...])
    for b in range(_BB):
        sm = smoother[:, b * nb:(b + 1) * nb]
        xb = x_t[:, b * nb:(b + 1) * nb]
        smooth = jnp.exp(-a * jnp.log(_EPS + sm))
        out_ref[b, 0] = jnp.exp(rr * jnp.log(xb * smooth + d)) - drr


def kernel(x, alpha, delta, r):
    bsz, c, nb, t_len = x.shape
    a_mat, q_full, decay = _tables(nb)
    grid = (bsz // _BB, t_len // _W)
    return pl.pallas_call(
        functools.partial(_pcen_body, decay=decay),
        grid=grid,
        in_specs=[
            pl.BlockSpec((_BB, 1, nb, _W), lambda b, t: (b, 0, 0, t)),
            pl.BlockSpec(a_mat.shape, lambda b, t: (0, 0)),
            pl.BlockSpec(q_full.shape, lambda b, t: (0, 0)),
            pl.BlockSpec((1, nb), lambda b, t: (0, 0)),
            pl.BlockSpec((1, nb), lambda b, t: (0, 0)),
            pl.BlockSpec((1, nb), lambda b, t: (0, 0)),
        ],
        out_specs=pl.BlockSpec((_BB, 1, _W, nb), lambda b, t: (b, 0, t, 0)),
        out_shape=jax.ShapeDtypeStruct((bsz, c, t_len, nb), x.dtype),
        scratch_shapes=[pltpu.VMEM((1, _BB * nb), jnp.float32)],
        compiler_params=pltpu.CompilerParams(
            dimension_semantics=("parallel", "arbitrary")),
    )(x, a_mat, q_full,
      alpha.reshape(1, nb), delta.reshape(1, nb), r.reshape(1, nb))


# speedup vs baseline: 3.9734x; 1.4059x over previous
import functools

import numpy as np
import jax
import jax.numpy as jnp
from jax.experimental import pallas as pl
from jax.experimental.pallas import tpu as pltpu

_T_VAL = 256.0
_EPS = 1e-05
_W = 512
_BB = 2


def _smoothing_coef() -> float:
    return float((np.sqrt(1.0 + 4.0 * _T_VAL ** 2) - 1.0) / (2.0 * _T_VAL ** 2))


@functools.lru_cache(maxsize=None)
def _tables(n_bands: int):
    s = _smoothing_coef()
    j = np.arange(_W)
    m = j[:, None] - j[None, :]
    vt = np.where(m >= 0, s * (1.0 - s) ** np.maximum(m, 0), 0.0)
    a_mat = np.concatenate(
        [vt, np.eye(_W), vt[-1:, :], np.zeros((7, _W))], axis=0)
    q_full = np.tile(((1.0 - s) ** (j + 1))[:, None], (1, _BB * n_bands))
    return (jnp.asarray(a_mat, dtype=jnp.float32),
            jnp.asarray(q_full, dtype=jnp.float32),
            float((1.0 - s) ** _W))


def _pcen_body(x_ref, a_ref, q_ref, alpha_ref, delta_ref, r_ref,
               out_ref, carry_ref, *, decay):
    t = pl.program_id(1)
    nb = x_ref.shape[2]
    w = q_ref.shape[0]
    x2 = x_ref[...].reshape(_BB * nb, w)
    res = jax.lax.dot_general(
        a_ref[...], x2, (((1,), (1,)), ((), ())),
        preferred_element_type=jnp.float32)
    x_t = res[w:2 * w, :]

    @pl.when(t == 0)
    def _():
        carry_ref[...] = x_t[0:1, :]

    carry = carry_ref[...]
    smoother = res[:w, :] + q_ref[...] * carry
    carry_ref[...] = res[2 * w:2 * w + 1, :] + decay * carry

    a = jnp.exp(alpha_ref[...])
    d = jnp.exp(delta_ref[...])
    rr = jnp.exp(r_ref[...])
    drr = jnp.exp(rr * delta_ref[...])
    for b in range(_BB):
        sm = smoother[:, b * nb:(b + 1) * nb]
        xb = x_t[:, b * nb:(b + 1) * nb]
        smooth = jnp.exp(-a * jnp.log(_EPS + sm))
        out_ref[b, 0] = jnp.exp(rr * jnp.log(xb * smooth + d)) - drr


def kernel(x, alpha, delta, r):
    bsz, c, nb, t_len = x.shape
    a_mat, q_full, decay = _tables(nb)
    grid = (bsz // _BB, t_len // _W)
    return pl.pallas_call(
        functools.partial(_pcen_body, decay=decay),
        grid=grid,
        in_specs=[
            pl.BlockSpec((_BB, 1, nb, _W), lambda b, t: (b, 0, 0, t)),
            pl.BlockSpec(a_mat.shape, lambda b, t: (0, 0)),
            pl.BlockSpec(q_full.shape, lambda b, t: (0, 0)),
            pl.BlockSpec((1, nb), lambda b, t: (0, 0)),
            pl.BlockSpec((1, nb), lambda b, t: (0, 0)),
            pl.BlockSpec((1, nb), lambda b, t: (0, 0)),
        ],
        out_specs=pl.BlockSpec((_BB, 1, _W, nb), lambda b, t: (b, 0, t, 0)),
        out_shape=jax.ShapeDtypeStruct((bsz, c, t_len, nb), x.dtype),
        scratch_shapes=[pltpu.VMEM((1, _BB * nb), jnp.float32)],
        compiler_params=pltpu.CompilerParams(
            dimension_semantics=("parallel", "arbitrary")),
    )(x, a_mat, q_full,
      alpha.reshape(1, nb), delta.reshape(1, nb), r.reshape(1, nb))


# speedup vs baseline: 5.2066x; 1.3104x over previous
import functools

import numpy as np
import jax
import jax.numpy as jnp
from jax.experimental import pallas as pl
from jax.experimental.pallas import tpu as pltpu

_T_VAL = 256.0
_EPS = 1e-05
_W = 512
_BB = 4


def _smoothing_coef() -> float:
    return float((np.sqrt(1.0 + 4.0 * _T_VAL ** 2) - 1.0) / (2.0 * _T_VAL ** 2))


@functools.lru_cache(maxsize=None)
def _tables(n_bands: int):
    s = _smoothing_coef()
    j = np.arange(_W)
    m = j[:, None] - j[None, :]
    vt = np.where(m >= 0, s * (1.0 - s) ** np.maximum(m, 0), 0.0)
    a_mat = np.concatenate(
        [vt, np.eye(_W), vt[-1:, :], np.zeros((7, _W))], axis=0)
    q_full = np.tile(((1.0 - s) ** (j + 1))[:, None], (1, _BB * n_bands))
    return (jnp.asarray(a_mat, dtype=jnp.float32),
            jnp.asarray(q_full, dtype=jnp.float32),
            float((1.0 - s) ** _W))


def _pcen_body(x_ref, a_ref, q_ref, alpha_ref, delta_ref, r_ref,
               out_ref, carry_ref, *, decay):
    t = pl.program_id(1)
    nb = x_ref.shape[2]
    w = q_ref.shape[0]
    x2 = x_ref[...].reshape(_BB * nb, w)
    res = jax.lax.dot_general(
        a_ref[...], x2, (((1,), (1,)), ((), ())),
        preferred_element_type=jnp.float32)
    x_t = res[w:2 * w, :]

    @pl.when(t == 0)
    def _():
        carry_ref[...] = x_t[0:1, :]

    carry = carry_ref[...]
    smoother = res[:w, :] + q_ref[...] * carry
    carry_ref[...] = res[2 * w:2 * w + 1, :] + decay * carry

    a = jnp.exp(alpha_ref[...])
    d = jnp.exp(delta_ref[...])
    rr = jnp.exp(r_ref[...])
    drr = jnp.exp(rr * delta_ref[...])
    for b in range(_BB):
        sm = smoother[:, b * nb:(b + 1) * nb]
        xb = x_t[:, b * nb:(b + 1) * nb]
        smooth = jnp.exp(-a * jnp.log(_EPS + sm))
        out_ref[b, 0] = jnp.exp(rr * jnp.log(xb * smooth + d)) - drr


def kernel(x, alpha, delta, r):
    bsz, c, nb, t_len = x.shape
    a_mat, q_full, decay = _tables(nb)
    grid = (bsz // _BB, t_len // _W)
    return pl.pallas_call(
        functools.partial(_pcen_body, decay=decay),
        grid=grid,
        in_specs=[
            pl.BlockSpec((_BB, 1, nb, _W), lambda b, t: (b, 0, 0, t)),
            pl.BlockSpec(a_mat.shape, lambda b, t: (0, 0)),
            pl.BlockSpec(q_full.shape, lambda b, t: (0, 0)),
            pl.BlockSpec((1, nb), lambda b, t: (0, 0)),
            pl.BlockSpec((1, nb), lambda b, t: (0, 0)),
            pl.BlockSpec((1, nb), lambda b, t: (0, 0)),
        ],
        out_specs=pl.BlockSpec((_BB, 1, _W, nb), lambda b, t: (b, 0, t, 0)),
        out_shape=jax.ShapeDtypeStruct((bsz, c, t_len, nb), x.dtype),
        scratch_shapes=[pltpu.VMEM((1, _BB * nb), jnp.float32)],
        compiler_params=pltpu.CompilerParams(
            dimension_semantics=("parallel", "arbitrary")),
    )(x, a_mat, q_full,
      alpha.reshape(1, nb), delta.reshape(1, nb), r.reshape(1, nb))


# speedup vs baseline: 5.9115x; 1.1354x over previous
import functools

import numpy as np
import jax
import jax.numpy as jnp
from jax.experimental import pallas as pl
from jax.experimental.pallas import tpu as pltpu

_T_VAL = 256.0
_EPS = 1e-05
_W = 512
_BB = 8


def _smoothing_coef() -> float:
    return float((np.sqrt(1.0 + 4.0 * _T_VAL ** 2) - 1.0) / (2.0 * _T_VAL ** 2))


@functools.lru_cache(maxsize=None)
def _tables(n_bands: int):
    s = _smoothing_coef()
    j = np.arange(_W)
    m = j[:, None] - j[None, :]
    vt = np.where(m >= 0, s * (1.0 - s) ** np.maximum(m, 0), 0.0)
    a_mat = np.concatenate(
        [vt, np.eye(_W), vt[-1:, :], np.zeros((7, _W))], axis=0)
    q_full = np.tile(((1.0 - s) ** (j + 1))[:, None], (1, _BB * n_bands))
    return (jnp.asarray(a_mat, dtype=jnp.float32),
            jnp.asarray(q_full, dtype=jnp.float32),
            float((1.0 - s) ** _W))


def _pcen_body(x_ref, a_ref, q_ref, alpha_ref, delta_ref, r_ref,
               out_ref, carry_ref, *, decay):
    t = pl.program_id(1)
    nb = x_ref.shape[2]
    w = q_ref.shape[0]
    x2 = x_ref[...].reshape(_BB * nb, w)
    res = jax.lax.dot_general(
        a_ref[...], x2, (((1,), (1,)), ((), ())),
        preferred_element_type=jnp.float32)
    x_t = res[w:2 * w, :]

    @pl.when(t == 0)
    def _():
        carry_ref[...] = x_t[0:1, :]

    carry = carry_ref[...]
    smoother = res[:w, :] + q_ref[...] * carry
    carry_ref[...] = res[2 * w:2 * w + 1, :] + decay * carry

    a = jnp.exp(alpha_ref[...])
    d = jnp.exp(delta_ref[...])
    rr = jnp.exp(r_ref[...])
    drr = jnp.exp(rr * delta_ref[...])
    for b in range(_BB):
        sm = smoother[:, b * nb:(b + 1) * nb]
        xb = x_t[:, b * nb:(b + 1) * nb]
        smooth = jnp.exp(-a * jnp.log(_EPS + sm))
        out_ref[b, 0] = jnp.exp(rr * jnp.log(xb * smooth + d)) - drr


def kernel(x, alpha, delta, r):
    bsz, c, nb, t_len = x.shape
    a_mat, q_full, decay = _tables(nb)
    grid = (bsz // _BB, t_len // _W)
    return pl.pallas_call(
        functools.partial(_pcen_body, decay=decay),
        grid=grid,
        in_specs=[
            pl.BlockSpec((_BB, 1, nb, _W), lambda b, t: (b, 0, 0, t)),
            pl.BlockSpec(a_mat.shape, lambda b, t: (0, 0)),
            pl.BlockSpec(q_full.shape, lambda b, t: (0, 0)),
            pl.BlockSpec((1, nb), lambda b, t: (0, 0)),
            pl.BlockSpec((1, nb), lambda b, t: (0, 0)),
            pl.BlockSpec((1, nb), lambda b, t: (0, 0)),
        ],
        out_specs=pl.BlockSpec((_BB, 1, _W, nb), lambda b, t: (b, 0, t, 0)),
        out_shape=jax.ShapeDtypeStruct((bsz, c, t_len, nb), x.dtype),
        scratch_shapes=[pltpu.VMEM((1, _BB * nb), jnp.float32)],
        compiler_params=pltpu.CompilerParams(
            dimension_semantics=("parallel", "arbitrary")),
    )(x, a_mat, q_full,
      alpha.reshape(1, nb), delta.reshape(1, nb), r.reshape(1, nb))
